# Initial kernel scaffold; baseline (speedup 1.0000x reference)
import jax
import jax.numpy as jnp
from jax.experimental import pallas as pl


def kernel(x, conv_weight):
    raise NotImplementedError("write your pallas kernel here")


# speedup vs baseline: 1.3773x; 1.3773x over previous
import math

import jax
import jax.numpy as jnp
from jax.experimental import pallas as pl
from jax.experimental.pallas import tpu as pltpu


def _band_from_taps(conv_weight: jnp.ndarray, channels: int, scale: float):
    k = conv_weight.shape[0]
    pad = (k - 1) // 2
    c = jnp.arange(channels)
    tap = c[None, :] - c[:, None] + pad
    valid = (tap >= 0) & (tap < k)
    w = jnp.where(valid, jnp.take(conv_weight, jnp.clip(tap, 0, k - 1)), 0.0)
    return (w * scale).astype(jnp.float32)


def _make_eca_body(batch_block: int, channels: int, hw: int):
    def body(x_ref, band_ref, o_ref):
        x = x_ref[...]
        flat = x.reshape(batch_block * channels, hw)
        ones = jnp.ones((hw, 1), dtype=jnp.float32)
        sums = jnp.dot(flat, ones,
                       preferred_element_type=jnp.float32)
        s = sums.reshape(batch_block, channels, 1)
        band = band_ref[...]
        for b in range(batch_block):
            conv = jnp.dot(band, s[b],
                           preferred_element_type=jnp.float32)
            att = 1.0 / (1.0 + jnp.exp(-conv))
            o_ref[b] = x[b] * att.astype(x.dtype)
    return body


def kernel(x, conv_weight):
    N, C, H, W = x.shape
    HW = H * W
    x3 = x.reshape(N, C, HW)
    band = _band_from_taps(conv_weight, C, 1.0 / float(HW))

    B = 8 if N % 8 == 0 else 1
    grid = N // B

    out3 = pl.pallas_call(
        _make_eca_body(B, C, HW),
        out_shape=jax.ShapeDtypeStruct((N, C, HW), x.dtype),
        grid_spec=pl.GridSpec(
            grid=(grid,),
            in_specs=[pl.BlockSpec((B, C, HW), lambda i: (i, 0, 0)),
                      pl.BlockSpec((C, C), lambda i: (0, 0))],
            out_specs=pl.BlockSpec((B, C, HW), lambda i: (i, 0, 0)),
        ),
        compiler_params=pltpu.CompilerParams(
            dimension_semantics=("parallel",),
            vmem_limit_bytes=48 << 20,
        ),
        cost_estimate=pl.CostEstimate(
            flops=2 * N * C * HW + 2 * N * C * C,
            transcendentals=N * C,
            bytes_accessed=2 * N * C * HW * 4 + C * C * 4,
        ),
    )(x3, band)
    return out3.reshape(N, C, H, W)


# speedup vs baseline: 1.4034x; 1.0189x over previous
import math

import jax
import jax.numpy as jnp
from jax.experimental import pallas as pl
from jax.experimental.pallas import tpu as pltpu


def _band_from_taps(conv_weight: jnp.ndarray, channels: int, scale: float):
    k = conv_weight.shape[0]
    pad = (k - 1) // 2
    c = jnp.arange(channels)
    tap = c[None, :] - c[:, None] + pad
    valid = (tap >= 0) & (tap < k)
    w = jnp.where(valid, jnp.take(conv_weight, jnp.clip(tap, 0, k - 1)), 0.0)
    return (w * scale).astype(jnp.float32)


def _make_eca_body(batch_block: int, channels: int, hw: int):
    def body(x_ref, band_ref, o_ref):
        x = x_ref[...]
        flat = x.reshape(batch_block * channels, hw)
        ones = jnp.ones((hw, 1), dtype=jnp.float32)
        sums = jnp.dot(flat, ones,
                       preferred_element_type=jnp.float32)
        s = sums.reshape(batch_block, channels, 1)
        band = band_ref[...]
        for b in range(batch_block):
            conv = jnp.dot(band, s[b],
                           preferred_element_type=jnp.float32)
            att = 1.0 / (1.0 + jnp.exp(-conv))
            o_ref[b] = x[b] * att.astype(x.dtype)
    return body


def kernel(x, conv_weight):
    N, C, H, W = x.shape
    HW = H * W
    x3 = x.reshape(N, C, HW)
    band = _band_from_taps(conv_weight, C, 1.0 / float(HW))

    B = 16 if N % 16 == 0 else 1
    grid = N // B

    out3 = pl.pallas_call(
        _make_eca_body(B, C, HW),
        out_shape=jax.ShapeDtypeStruct((N, C, HW), x.dtype),
        grid_spec=pl.GridSpec(
            grid=(grid,),
            in_specs=[pl.BlockSpec((B, C, HW), lambda i: (i, 0, 0)),
                      pl.BlockSpec((C, C), lambda i: (0, 0))],
            out_specs=pl.BlockSpec((B, C, HW), lambda i: (i, 0, 0)),
        ),
        compiler_params=pltpu.CompilerParams(
            dimension_semantics=("parallel",),
            vmem_limit_bytes=48 << 20,
        ),
        cost_estimate=pl.CostEstimate(
            flops=2 * N * C * HW + 2 * N * C * C,
            transcendentals=N * C,
            bytes_accessed=2 * N * C * HW * 4 + C * C * 4,
        ),
    )(x3, band)
    return out3.reshape(N, C, H, W)
